# baseline (speedup 1.0000x reference)
import functools

import jax
import jax.numpy as jnp
from jax import lax
from jax.experimental import pallas as pl
from jax.experimental.pallas import tpu as pltpu
from jax.experimental.pallas import tpu_sc as plsc

N_NODES = 10000
E_EDGES = 320000
C_IN, C_HID, C_OUT = 128, 32, 8

NC, NS = 2, 16
NW = NC * NS
NPAD = 10240
RPS = NPAD // NS
CHUNK = 128
CPW = 80
EPAD = NW * CPW * CHUNK
EROWS = EPAD // CHUNK
ZROWS = 128
DW = 16
NB = 8

_MESH = plsc.VectorSubcoreMesh(
    core_axis_name="c", subcore_axis_name="s", num_cores=NC, num_subcores=NS
)
_SC_PARAMS = pltpu.CompilerParams(use_tc_tiling_on_sc=False)


def _fill2d(buf, rows, cols, value):
    segs = cols // 16

    def body(k, _):
        buf[k // segs, pl.ds((k % segs) * 16, 16)] = jnp.full(
            (16,), value, jnp.float32
        )
        return 0

    lax.fori_loop(0, rows * segs, body, 0)


@functools.partial(
    pl.kernel,
    out_type=[jax.ShapeDtypeStruct((NPAD, C_HID), jnp.float32)] * NC,
    mesh=_MESH,
    scratch_types=[
        pltpu.VMEM_SHARED((NPAD, C_HID), jnp.float32),
        pltpu.VMEM_SHARED((NPAD, C_HID), jnp.float32),
        pltpu.VMEM((ZROWS, C_HID), jnp.float32),
        pltpu.VMEM((CPW, CHUNK), jnp.int32),
        pltpu.VMEM((CPW, CHUNK), jnp.int32),
        pltpu.VMEM((NB, CHUNK, C_HID), jnp.float32),
        pltpu.SemaphoreType.DMA,
        pltpu.SemaphoreType.DMA,
        pltpu.SemaphoreType.DMA,
        pltpu.SemaphoreType.DMA,
        pltpu.SemaphoreType.DMA,
        pltpu.SemaphoreType.DMA,
        pltpu.SemaphoreType.DMA,
        pltpu.SemaphoreType.DMA,
        pltpu.SemaphoreType.DMA,
        pltpu.SemaphoreType.DMA,
        pltpu.SemaphoreType.DMA,
        pltpu.SemaphoreType.DMA,
        pltpu.SemaphoreType.DMA,
        pltpu.SemaphoreType.DMA,
        pltpu.SemaphoreType.DMA,
        pltpu.SemaphoreType.DMA,
    ],
    compiler_params=_SC_PARAMS,
)
def _sc_agg(table, srch, dsth, outa, outb, acc, tbl, zrows, sidx, didx, rows,
            *sems):
    c = lax.axis_index("c")
    s = lax.axis_index("s")
    wid = s * NC + c
    gsem = sems[:NB]
    tsem = sems[NB:]
    LEAD = 4

    _fill2d(zrows, ZROWS, C_HID, 0.0)
    for k in range(RPS // ZROWS):
        pltpu.sync_copy(zrows, acc.at[pl.ds(s * RPS + k * ZROWS, ZROWS)])
    pltpu.sync_copy(srch.at[pl.ds(wid * CPW, CPW)], sidx)
    pltpu.sync_copy(dsth.at[pl.ds(wid * CPW, CPW)], didx)
    pltpu.sync_copy(table.at[pl.ds(s * RPS, RPS)], tbl.at[pl.ds(s * RPS, RPS)])
    plsc.subcore_barrier()

    def _wait_gather(b):
        pltpu.make_async_copy(tbl.at[sidx.at[0]], rows.at[b],
                              gsem[b]).wait()

    def _wait_scatter(b):
        pltpu.make_async_copy(rows.at[b], acc.at[didx.at[0]],
                              tsem[b]).wait()

    def _chunk(j, jj):
        b = j % NB
        _wait_gather(b)
        pltpu.async_copy(rows.at[b], acc.at[didx.at[jj]], tsem[b], add=True)

    for b in range(LEAD):
        pltpu.async_copy(tbl.at[sidx.at[b]], rows.at[b], gsem[b])

    for j in range(NB):
        _chunk(j, j)
        nxt = j + LEAD
        if nxt < NB:
            pass
        else:
            _wait_scatter(nxt % NB)
        pltpu.async_copy(tbl.at[sidx.at[nxt]], rows.at[nxt % NB],
                         gsem[nxt % NB])

    def body(g, _):
        for b in range(NB):
            j = NB * g + b
            _chunk(b, j)
            b2 = (b + LEAD) % NB
            _wait_scatter(b2)
            pltpu.async_copy(tbl.at[sidx.at[j + LEAD]], rows.at[b2],
                             gsem[b2])
        return 0

    lax.fori_loop(1, CPW // NB - 1, body, 0)

    for j in range(CPW - NB, CPW):
        _chunk(j % NB, j)
        nxt = j + LEAD
        if nxt < CPW:
            _wait_scatter(nxt % NB)
            pltpu.async_copy(tbl.at[sidx.at[nxt]], rows.at[nxt % NB],
                             gsem[nxt % NB])

    for b in range(NB):
        _wait_scatter(b)

    plsc.subcore_barrier()

    @pl.when(c == 0)
    def _():
        pltpu.sync_copy(acc.at[pl.ds(s * RPS, RPS)], outa.at[pl.ds(s * RPS, RPS)])

    @pl.when(c == 1)
    def _():
        pltpu.sync_copy(acc.at[pl.ds(s * RPS, RPS)], outb.at[pl.ds(s * RPS, RPS)])


@functools.partial(
    pl.kernel,
    out_type=[jax.ShapeDtypeStruct((NPAD, DW), jnp.float32)] * NC,
    mesh=_MESH,
    scratch_types=[
        pltpu.VMEM_SHARED((NPAD, DW), jnp.float32),
        pltpu.VMEM((ZROWS, DW), jnp.float32),
        pltpu.VMEM((CHUNK, DW), jnp.float32),
        pltpu.VMEM((CPW, CHUNK), jnp.int32),
    ],
    compiler_params=_SC_PARAMS,
)
def _sc_deg(dsth, outa, outb, dacc, zrows, ones, didx):
    c = lax.axis_index("c")
    s = lax.axis_index("s")
    wid = s * NC + c

    _fill2d(zrows, ZROWS, DW, 0.0)
    _fill2d(ones, CHUNK, DW, 1.0)
    for k in range(RPS // ZROWS):
        pltpu.sync_copy(zrows, dacc.at[pl.ds(s * RPS + k * ZROWS, ZROWS)])
    pltpu.sync_copy(dsth.at[pl.ds(wid * CPW, CPW)], didx)
    plsc.subcore_barrier()

    def body(j, _):
        pltpu.sync_copy(ones, dacc.at[didx.at[j]], add=True)
        return 0

    lax.fori_loop(0, CPW, body, 0)

    plsc.subcore_barrier()

    @pl.when(c == 0)
    def _():
        pltpu.sync_copy(dacc.at[pl.ds(s * RPS, RPS)], outa.at[pl.ds(s * RPS, RPS)])

    @pl.when(c == 1)
    def _():
        pltpu.sync_copy(dacc.at[pl.ds(s * RPS, RPS)], outb.at[pl.ds(s * RPS, RPS)])


BR = 2048


def _stage_a_body(x_ref, wl_ref, wr_ref, y_ref, r_ref):
    xb = x_ref[...]
    y_ref[...] = jnp.dot(xb, wl_ref[...], preferred_element_type=jnp.float32)
    r_ref[...] = jnp.dot(xb, wr_ref[...], preferred_element_type=jnp.float32)


def _stage_b_body(pa_ref, pb_ref, da_ref, db_ref, bl_ref, r_ref, wl_ref,
                  wr_ref, y_ref, rn_ref, inv_ref):
    dsum = da_ref[...][:, 0:1] + db_ref[...][:, 0:1]
    inv = 1.0 / jnp.maximum(dsum, 1.0)
    h = jnp.maximum(
        (pa_ref[...] + pb_ref[...]) * inv + bl_ref[0:1, :] + r_ref[...], 0.0
    )
    y_ref[...] = jnp.dot(h, wl_ref[...], preferred_element_type=jnp.float32)
    rn_ref[...] = jnp.dot(h, wr_ref[...], preferred_element_type=jnp.float32)
    inv_ref[...] = jnp.broadcast_to(inv, (BR, C_HID))


def _stage_c_body(pa_ref, pb_ref, inv_ref, bl_ref, r_ref, wl_ref, wr_ref,
                  y_ref, rn_ref):
    h = jnp.maximum(
        (pa_ref[...] + pb_ref[...]) * inv_ref[...] + bl_ref[0:1, :]
        + r_ref[...], 0.0
    )
    y_ref[...] = jnp.dot(h, wl_ref[...], preferred_element_type=jnp.float32)
    rn_ref[...] = jnp.dot(h, wr_ref[...], preferred_element_type=jnp.float32)


def _stage_d_body(pa_ref, pb_ref, inv_ref, bl_ref, r_ref, w4_ref, b4_ref,
                  o_ref):
    h = jnp.maximum(
        (pa_ref[...] + pb_ref[...]) * inv_ref[...] + bl_ref[0:1, :]
        + r_ref[...], 0.0
    )
    o_ref[...] = (
        jnp.dot(h, w4_ref[...], preferred_element_type=jnp.float32)
        + b4_ref[0:1, :]
    )


def _rows(bs):
    return pl.BlockSpec((BR, bs), lambda i: (i, 0))


def _full(a, b):
    return pl.BlockSpec((a, b), lambda i: (0, 0))


def kernel(x, edge_index, Wl1, bl1, Wr1, Wl2, bl2, Wr2, Wl3, bl3, Wr3, W4, b4):
    xp = jnp.concatenate(
        [x, jnp.zeros((NPAD - N_NODES, C_IN), jnp.float32)], axis=0
    )
    epad = EPAD - E_EDGES
    srcp = jnp.concatenate(
        [edge_index[0], jnp.zeros((epad,), jnp.int32)]
    ).reshape(EROWS, CHUNK)
    dstp = jnp.concatenate(
        [edge_index[1], jnp.full((epad,), NPAD - 1, jnp.int32)]
    ).reshape(EROWS, CHUNK)

    bl1b = jnp.broadcast_to(bl1[None, :], (8, C_HID))
    bl2b = jnp.broadcast_to(bl2[None, :], (8, C_HID))
    bl3b = jnp.broadcast_to(bl3[None, :], (8, C_HID))
    b4b = jnp.broadcast_to(b4[None, :], (8, C_OUT))

    da, db = _sc_deg(dstp)

    y1, r1 = pl.pallas_call(
        _stage_a_body,
        grid=(NPAD // BR,),
        in_specs=[_rows(C_IN), _full(C_IN, C_HID), _full(C_IN, C_HID)],
        out_specs=[_rows(C_HID), _rows(C_HID)],
        out_shape=[jax.ShapeDtypeStruct((NPAD, C_HID), jnp.float32)] * 2,
    )(xp, Wl1, Wr1)

    a1, b1 = _sc_agg(y1, srcp, dstp)
    y2, r2, invd = pl.pallas_call(
        _stage_b_body,
        grid=(NPAD // BR,),
        in_specs=[_rows(C_HID), _rows(C_HID), _rows(DW), _rows(DW),
                  _full(8, C_HID), _rows(C_HID), _full(C_HID, C_HID),
                  _full(C_HID, C_HID)],
        out_specs=[_rows(C_HID), _rows(C_HID), _rows(C_HID)],
        out_shape=[jax.ShapeDtypeStruct((NPAD, C_HID), jnp.float32)] * 3,
    )(a1, b1, da, db, bl1b, r1, Wl2, Wr2)

    a2, b2 = _sc_agg(y2, srcp, dstp)
    y3, r3 = pl.pallas_call(
        _stage_c_body,
        grid=(NPAD // BR,),
        in_specs=[_rows(C_HID), _rows(C_HID), _rows(C_HID), _full(8, C_HID),
                  _rows(C_HID), _full(C_HID, C_HID), _full(C_HID, C_HID)],
        out_specs=[_rows(C_HID), _rows(C_HID)],
        out_shape=[jax.ShapeDtypeStruct((NPAD, C_HID), jnp.float32)] * 2,
    )(a2, b2, invd, bl2b, r2, Wl3, Wr3)

    a3, b3 = _sc_agg(y3, srcp, dstp)
    out = pl.pallas_call(
        _stage_d_body,
        grid=(NPAD // BR,),
        in_specs=[_rows(C_HID), _rows(C_HID), _rows(C_HID), _full(8, C_HID),
                  _rows(C_HID), _full(C_HID, C_OUT), _full(8, C_OUT)],
        out_specs=_rows(C_OUT),
        out_shape=jax.ShapeDtypeStruct((NPAD, C_OUT), jnp.float32),
    )(a3, b3, invd, bl3b, r3, W4, b4b)

    return out[:N_NODES - 1]


# speedup vs baseline: 1.0034x; 1.0034x over previous
import functools

import jax
import jax.numpy as jnp
from jax import lax
from jax.experimental import pallas as pl
from jax.experimental.pallas import tpu as pltpu
from jax.experimental.pallas import tpu_sc as plsc

N_NODES = 10000
E_EDGES = 320000
C_IN, C_HID, C_OUT = 128, 32, 8

NC, NS = 2, 16
NW = NC * NS
NPAD = 10240
RPS = NPAD // NS
CHUNK = 128
CPW = 80
EPAD = NW * CPW * CHUNK
EROWS = EPAD // CHUNK
ZROWS = 128
DW = 16
NB = 8

_MESH = plsc.VectorSubcoreMesh(
    core_axis_name="c", subcore_axis_name="s", num_cores=NC, num_subcores=NS
)
_SC_PARAMS = pltpu.CompilerParams(use_tc_tiling_on_sc=False)


def _fill2d(buf, rows, cols, value):
    segs = cols // 16

    def body(k, _):
        buf[k // segs, pl.ds((k % segs) * 16, 16)] = jnp.full(
            (16,), value, jnp.float32
        )
        return 0

    lax.fori_loop(0, rows * segs, body, 0)


@functools.partial(
    pl.kernel,
    out_type=[jax.ShapeDtypeStruct((NPAD, C_HID), jnp.float32)] * NC,
    mesh=_MESH,
    scratch_types=[
        pltpu.VMEM_SHARED((NPAD, C_HID), jnp.float32),
        pltpu.VMEM_SHARED((NPAD, C_HID), jnp.float32),
        pltpu.VMEM((ZROWS, C_HID), jnp.float32),
        pltpu.VMEM((CPW, CHUNK), jnp.int32),
        pltpu.VMEM((CPW, CHUNK), jnp.int32),
        pltpu.VMEM((NB, CHUNK, C_HID), jnp.float32),
        pltpu.SemaphoreType.DMA,
        pltpu.SemaphoreType.DMA,
        pltpu.SemaphoreType.DMA,
        pltpu.SemaphoreType.DMA,
        pltpu.SemaphoreType.DMA,
        pltpu.SemaphoreType.DMA,
        pltpu.SemaphoreType.DMA,
        pltpu.SemaphoreType.DMA,
        pltpu.SemaphoreType.DMA,
        pltpu.SemaphoreType.DMA,
        pltpu.SemaphoreType.DMA,
        pltpu.SemaphoreType.DMA,
        pltpu.SemaphoreType.DMA,
        pltpu.SemaphoreType.DMA,
        pltpu.SemaphoreType.DMA,
        pltpu.SemaphoreType.DMA,
    ],
    compiler_params=_SC_PARAMS,
)
def _sc_agg(table, srch, dsth, outa, outb, acc, tbl, zrows, sidx, didx, rows,
            *sems):
    c = lax.axis_index("c")
    s = lax.axis_index("s")
    wid = s * NC + c
    gsem = sems[:NB]
    tsem = sems[NB:]
    LEAD = 4

    _fill2d(zrows, ZROWS, C_HID, 0.0)
    for k in range(RPS // ZROWS):
        pltpu.sync_copy(zrows, acc.at[pl.ds(s * RPS + k * ZROWS, ZROWS)])
    pltpu.sync_copy(srch.at[pl.ds(wid * CPW, CPW)], sidx)
    pltpu.sync_copy(dsth.at[pl.ds(wid * CPW, CPW)], didx)
    pltpu.sync_copy(table.at[pl.ds(s * RPS, RPS)], tbl.at[pl.ds(s * RPS, RPS)])
    plsc.subcore_barrier()

    def _wait_gather(b):
        pltpu.make_async_copy(tbl.at[sidx.at[0]], rows.at[b],
                              gsem[b]).wait()

    def _wait_scatter(b):
        pltpu.make_async_copy(rows.at[b], acc.at[didx.at[0]],
                              tsem[b]).wait()

    def _chunk(j, jj):
        b = j % NB
        _wait_gather(b)
        pltpu.async_copy(rows.at[b], acc.at[didx.at[jj]], tsem[b], add=True)

    for b in range(LEAD):
        pltpu.async_copy(tbl.at[sidx.at[b]], rows.at[b], gsem[b])

    for j in range(NB):
        _chunk(j, j)
        nxt = j + LEAD
        if nxt < NB:
            pass
        else:
            _wait_scatter(nxt % NB)
        pltpu.async_copy(tbl.at[sidx.at[nxt]], rows.at[nxt % NB],
                         gsem[nxt % NB])

    def body(g, _):
        for b in range(NB):
            j = NB * g + b
            _chunk(b, j)
            b2 = (b + LEAD) % NB
            _wait_scatter(b2)
            pltpu.async_copy(tbl.at[sidx.at[j + LEAD]], rows.at[b2],
                             gsem[b2])
        return 0

    lax.fori_loop(1, CPW // NB - 1, body, 0)

    for j in range(CPW - NB, CPW):
        _chunk(j % NB, j)
        nxt = j + LEAD
        if nxt < CPW:
            _wait_scatter(nxt % NB)
            pltpu.async_copy(tbl.at[sidx.at[nxt]], rows.at[nxt % NB],
                             gsem[nxt % NB])

    for b in range(NB):
        _wait_scatter(b)

    plsc.subcore_barrier()

    @pl.when(c == 0)
    def _():
        pltpu.sync_copy(acc.at[pl.ds(s * RPS, RPS)], outa.at[pl.ds(s * RPS, RPS)])

    @pl.when(c == 1)
    def _():
        pltpu.sync_copy(acc.at[pl.ds(s * RPS, RPS)], outb.at[pl.ds(s * RPS, RPS)])


@functools.partial(
    pl.kernel,
    out_type=[jax.ShapeDtypeStruct((NPAD, DW), jnp.float32)] * NC,
    mesh=_MESH,
    scratch_types=[
        pltpu.VMEM_SHARED((NPAD, DW), jnp.float32),
        pltpu.VMEM((ZROWS, DW), jnp.float32),
        pltpu.VMEM((CHUNK, DW), jnp.float32),
        pltpu.VMEM((CPW, CHUNK), jnp.int32),
    ],
    compiler_params=_SC_PARAMS,
)
def _sc_deg(dsth, outa, outb, dacc, zrows, ones, didx):
    c = lax.axis_index("c")
    s = lax.axis_index("s")
    wid = s * NC + c

    _fill2d(zrows, ZROWS, DW, 0.0)
    _fill2d(ones, CHUNK, DW, 1.0)
    for k in range(RPS // ZROWS):
        pltpu.sync_copy(zrows, dacc.at[pl.ds(s * RPS + k * ZROWS, ZROWS)])
    pltpu.sync_copy(dsth.at[pl.ds(wid * CPW, CPW)], didx)
    plsc.subcore_barrier()

    def body(j, _):
        pltpu.sync_copy(ones, dacc.at[didx.at[j]], add=True)
        return 0

    lax.fori_loop(0, CPW, body, 0)

    plsc.subcore_barrier()

    @pl.when(c == 0)
    def _():
        pltpu.sync_copy(dacc.at[pl.ds(s * RPS, RPS)], outa.at[pl.ds(s * RPS, RPS)])

    @pl.when(c == 1)
    def _():
        pltpu.sync_copy(dacc.at[pl.ds(s * RPS, RPS)], outb.at[pl.ds(s * RPS, RPS)])


BR = 2048


def _stage_a_body(x_ref, wl_ref, wr_ref, da_ref, db_ref, y_ref, r_ref):
    xb = x_ref[...]
    y_ref[...] = jnp.dot(xb, wl_ref[...], preferred_element_type=jnp.float32)
    r_ref[...] = jnp.dot(xb, wr_ref[...], preferred_element_type=jnp.float32)


def _stage_b_body(pa_ref, pb_ref, da_ref, db_ref, bl_ref, r_ref, wl_ref,
                  wr_ref, y_ref, rn_ref, inv_ref):
    dsum = da_ref[...][:, 0:1] + db_ref[...][:, 0:1]
    inv = 1.0 / jnp.maximum(dsum, 1.0)
    h = jnp.maximum(
        (pa_ref[...] + pb_ref[...]) * inv + bl_ref[0:1, :] + r_ref[...], 0.0
    )
    y_ref[...] = jnp.dot(h, wl_ref[...], preferred_element_type=jnp.float32)
    rn_ref[...] = jnp.dot(h, wr_ref[...], preferred_element_type=jnp.float32)
    inv_ref[...] = jnp.broadcast_to(inv, (BR, C_HID))


def _stage_c_body(pa_ref, pb_ref, inv_ref, bl_ref, r_ref, wl_ref, wr_ref,
                  y_ref, rn_ref):
    h = jnp.maximum(
        (pa_ref[...] + pb_ref[...]) * inv_ref[...] + bl_ref[0:1, :]
        + r_ref[...], 0.0
    )
    y_ref[...] = jnp.dot(h, wl_ref[...], preferred_element_type=jnp.float32)
    rn_ref[...] = jnp.dot(h, wr_ref[...], preferred_element_type=jnp.float32)


def _stage_d_body(pa_ref, pb_ref, inv_ref, bl_ref, r_ref, w4_ref, b4_ref,
                  o_ref):
    h = jnp.maximum(
        (pa_ref[...] + pb_ref[...]) * inv_ref[...] + bl_ref[0:1, :]
        + r_ref[...], 0.0
    )
    o_ref[...] = (
        jnp.dot(h, w4_ref[...], preferred_element_type=jnp.float32)
        + b4_ref[0:1, :]
    )


def _rows(bs):
    return pl.BlockSpec((BR, bs), lambda i: (i, 0))


def _full(a, b):
    return pl.BlockSpec((a, b), lambda i: (0, 0))


def kernel(x, edge_index, Wl1, bl1, Wr1, Wl2, bl2, Wr2, Wl3, bl3, Wr3, W4, b4):
    xp = jnp.concatenate(
        [x, jnp.zeros((NPAD - N_NODES, C_IN), jnp.float32)], axis=0
    )
    epad = EPAD - E_EDGES
    srcp = jnp.concatenate(
        [edge_index[0], jnp.zeros((epad,), jnp.int32)]
    ).reshape(EROWS, CHUNK)
    dstp = jnp.concatenate(
        [edge_index[1], jnp.full((epad,), NPAD - 1, jnp.int32)]
    ).reshape(EROWS, CHUNK)

    bl1b = jnp.broadcast_to(bl1[None, :], (8, C_HID))
    bl2b = jnp.broadcast_to(bl2[None, :], (8, C_HID))
    bl3b = jnp.broadcast_to(bl3[None, :], (8, C_HID))
    b4b = jnp.broadcast_to(b4[None, :], (8, C_OUT))

    da, db = _sc_deg(dstp)

    y1, r1 = pl.pallas_call(
        _stage_a_body,
        grid=(NPAD // BR,),
        in_specs=[_rows(C_IN), _full(C_IN, C_HID), _full(C_IN, C_HID),
                  _full(8, DW), _full(8, DW)],
        out_specs=[_rows(C_HID), _rows(C_HID)],
        out_shape=[jax.ShapeDtypeStruct((NPAD, C_HID), jnp.float32)] * 2,
    )(xp, Wl1, Wr1, da, db)

    a1, b1 = _sc_agg(y1, srcp, dstp)
    y2, r2, invd = pl.pallas_call(
        _stage_b_body,
        grid=(NPAD // BR,),
        in_specs=[_rows(C_HID), _rows(C_HID), _rows(DW), _rows(DW),
                  _full(8, C_HID), _rows(C_HID), _full(C_HID, C_HID),
                  _full(C_HID, C_HID)],
        out_specs=[_rows(C_HID), _rows(C_HID), _rows(C_HID)],
        out_shape=[jax.ShapeDtypeStruct((NPAD, C_HID), jnp.float32)] * 3,
    )(a1, b1, da, db, bl1b, r1, Wl2, Wr2)

    a2, b2 = _sc_agg(y2, srcp, dstp)
    y3, r3 = pl.pallas_call(
        _stage_c_body,
        grid=(NPAD // BR,),
        in_specs=[_rows(C_HID), _rows(C_HID), _rows(C_HID), _full(8, C_HID),
                  _rows(C_HID), _full(C_HID, C_HID), _full(C_HID, C_HID)],
        out_specs=[_rows(C_HID), _rows(C_HID)],
        out_shape=[jax.ShapeDtypeStruct((NPAD, C_HID), jnp.float32)] * 2,
    )(a2, b2, invd, bl2b, r2, Wl3, Wr3)

    a3, b3 = _sc_agg(y3, srcp, dstp)
    out = pl.pallas_call(
        _stage_d_body,
        grid=(NPAD // BR,),
        in_specs=[_rows(C_HID), _rows(C_HID), _rows(C_HID), _full(8, C_HID),
                  _rows(C_HID), _full(C_HID, C_OUT), _full(8, C_OUT)],
        out_specs=_rows(C_OUT),
        out_shape=jax.ShapeDtypeStruct((NPAD, C_OUT), jnp.float32),
    )(a3, b3, invd, bl3b, r3, W4, b4b)

    return out[:N_NODES - 1]


# speedup vs baseline: 1.0817x; 1.0780x over previous
import functools

import jax
import jax.numpy as jnp
from jax import lax
from jax.experimental import pallas as pl
from jax.experimental.pallas import tpu as pltpu
from jax.experimental.pallas import tpu_sc as plsc

N_NODES = 10000
E_EDGES = 320000
C_IN, C_HID, C_OUT = 128, 32, 8

NC, NS = 2, 16
NW = NC * NS
NPAD = 10240
RPS = NPAD // NS
CHUNK = 128
CPW = 80
EPAD = NW * CPW * CHUNK
EROWS = EPAD // CHUNK
ZROWS = 128
DW = 16
NB = 8

_MESH = plsc.VectorSubcoreMesh(
    core_axis_name="c", subcore_axis_name="s", num_cores=NC, num_subcores=NS
)
_SC_PARAMS = pltpu.CompilerParams(use_tc_tiling_on_sc=False)


def _fill2d(buf, rows, cols, value):
    segs = cols // 16

    def body(k, _):
        buf[k // segs, pl.ds((k % segs) * 16, 16)] = jnp.full(
            (16,), value, jnp.float32
        )
        return 0

    lax.fori_loop(0, rows * segs, body, 0)


@functools.partial(
    pl.kernel,
    out_type=[jax.ShapeDtypeStruct((NPAD, C_HID), jnp.float32)] * NC,
    mesh=_MESH,
    scratch_types=[
        pltpu.VMEM_SHARED((NPAD, C_HID), jnp.float32),
        pltpu.VMEM_SHARED((NPAD, C_HID), jnp.float32),
        pltpu.VMEM((ZROWS, C_HID), jnp.float32),
        pltpu.VMEM((CPW, CHUNK), jnp.int32),
        pltpu.VMEM((CPW, CHUNK), jnp.int32),
        pltpu.VMEM((NB, CHUNK, C_HID), jnp.float32),
        pltpu.SemaphoreType.DMA,
        pltpu.SemaphoreType.DMA,
        pltpu.SemaphoreType.DMA,
        pltpu.SemaphoreType.DMA,
        pltpu.SemaphoreType.DMA,
        pltpu.SemaphoreType.DMA,
        pltpu.SemaphoreType.DMA,
        pltpu.SemaphoreType.DMA,
        pltpu.SemaphoreType.DMA,
        pltpu.SemaphoreType.DMA,
        pltpu.SemaphoreType.DMA,
        pltpu.SemaphoreType.DMA,
        pltpu.SemaphoreType.DMA,
        pltpu.SemaphoreType.DMA,
        pltpu.SemaphoreType.DMA,
        pltpu.SemaphoreType.DMA,
    ],
    compiler_params=_SC_PARAMS,
)
def _sc_agg(table, srch, dsth, dga, dgb, outa, outb, acc, tbl, zrows, sidx,
            didx, rows, *sems):
    c = lax.axis_index("c")
    s = lax.axis_index("s")
    wid = s * NC + c
    gsem = sems[:NB]
    tsem = sems[NB:]
    LEAD = 4

    _fill2d(zrows, ZROWS, C_HID, 0.0)
    for k in range(RPS // ZROWS):
        pltpu.sync_copy(zrows, acc.at[pl.ds(s * RPS + k * ZROWS, ZROWS)])
    pltpu.sync_copy(srch.at[pl.ds(wid * CPW, CPW)], sidx)
    pltpu.sync_copy(dsth.at[pl.ds(wid * CPW, CPW)], didx)
    pltpu.sync_copy(table.at[pl.ds(s * RPS, RPS)], tbl.at[pl.ds(s * RPS, RPS)])
    plsc.subcore_barrier()

    def _wait_gather(b):
        pltpu.make_async_copy(tbl.at[sidx.at[0]], rows.at[b],
                              gsem[b]).wait()

    def _wait_scatter(b):
        pltpu.make_async_copy(rows.at[b], acc.at[didx.at[0]],
                              tsem[b]).wait()

    def _chunk(j, jj):
        b = j % NB
        _wait_gather(b)
        pltpu.async_copy(rows.at[b], acc.at[didx.at[jj]], tsem[b], add=True)

    for b in range(LEAD):
        pltpu.async_copy(tbl.at[sidx.at[b]], rows.at[b], gsem[b])

    for j in range(NB):
        _chunk(j, j)
        nxt = j + LEAD
        if nxt < NB:
            pass
        else:
            _wait_scatter(nxt % NB)
        pltpu.async_copy(tbl.at[sidx.at[nxt]], rows.at[nxt % NB],
                         gsem[nxt % NB])

    def body(g, _):
        for b in range(NB):
            j = NB * g + b
            _chunk(b, j)
            b2 = (b + LEAD) % NB
            _wait_scatter(b2)
            pltpu.async_copy(tbl.at[sidx.at[j + LEAD]], rows.at[b2],
                             gsem[b2])
        return 0

    lax.fori_loop(1, CPW // NB - 1, body, 0)

    for j in range(CPW - NB, CPW):
        _chunk(j % NB, j)
        nxt = j + LEAD
        if nxt < CPW:
            _wait_scatter(nxt % NB)
            pltpu.async_copy(tbl.at[sidx.at[nxt]], rows.at[nxt % NB],
                             gsem[nxt % NB])

    for b in range(NB):
        _wait_scatter(b)

    plsc.subcore_barrier()

    @pl.when(c == 0)
    def _():
        pltpu.sync_copy(acc.at[pl.ds(s * RPS, RPS)], outa.at[pl.ds(s * RPS, RPS)])

    @pl.when(c == 1)
    def _():
        pltpu.sync_copy(acc.at[pl.ds(s * RPS, RPS)], outb.at[pl.ds(s * RPS, RPS)])


@functools.partial(
    pl.kernel,
    out_type=[jax.ShapeDtypeStruct((NPAD, DW), jnp.float32)] * NC,
    mesh=_MESH,
    scratch_types=[
        pltpu.VMEM_SHARED((NPAD, DW), jnp.float32),
        pltpu.VMEM((ZROWS, DW), jnp.float32),
        pltpu.VMEM((CHUNK, DW), jnp.float32),
        pltpu.VMEM((CPW, CHUNK), jnp.int32),
    ],
    compiler_params=_SC_PARAMS,
)
def _sc_deg(dsth, outa, outb, dacc, zrows, ones, didx):
    c = lax.axis_index("c")
    s = lax.axis_index("s")
    wid = s * NC + c

    _fill2d(zrows, ZROWS, DW, 0.0)
    _fill2d(ones, CHUNK, DW, 1.0)
    for k in range(RPS // ZROWS):
        pltpu.sync_copy(zrows, dacc.at[pl.ds(s * RPS + k * ZROWS, ZROWS)])
    pltpu.sync_copy(dsth.at[pl.ds(wid * CPW, CPW)], didx)
    plsc.subcore_barrier()

    def body(j, _):
        pltpu.sync_copy(ones, dacc.at[didx.at[j]], add=True)
        return 0

    lax.fori_loop(0, CPW, body, 0)

    plsc.subcore_barrier()

    @pl.when(c == 0)
    def _():
        pltpu.sync_copy(dacc.at[pl.ds(s * RPS, RPS)], outa.at[pl.ds(s * RPS, RPS)])

    @pl.when(c == 1)
    def _():
        pltpu.sync_copy(dacc.at[pl.ds(s * RPS, RPS)], outb.at[pl.ds(s * RPS, RPS)])


BR = 2048


def _stage_a_body(x_ref, wl_ref, wr_ref, y_ref, r_ref):
    xb = x_ref[...]
    y_ref[...] = jnp.dot(xb, wl_ref[...], preferred_element_type=jnp.float32)
    r_ref[...] = jnp.dot(xb, wr_ref[...], preferred_element_type=jnp.float32)


def _stage_b_body(pa_ref, pb_ref, da_ref, db_ref, bl_ref, r_ref, wl_ref,
                  wr_ref, y_ref, rn_ref, inv_ref):
    dsum = da_ref[...][:, 0:1] + db_ref[...][:, 0:1]
    inv = 1.0 / jnp.maximum(dsum, 1.0)
    h = jnp.maximum(
        (pa_ref[...] + pb_ref[...]) * inv + bl_ref[0:1, :] + r_ref[...], 0.0
    )
    y_ref[...] = jnp.dot(h, wl_ref[...], preferred_element_type=jnp.float32)
    rn_ref[...] = jnp.dot(h, wr_ref[...], preferred_element_type=jnp.float32)
    inv_ref[...] = jnp.broadcast_to(inv, (BR, C_HID))


def _stage_c_body(pa_ref, pb_ref, inv_ref, bl_ref, r_ref, wl_ref, wr_ref,
                  y_ref, rn_ref):
    h = jnp.maximum(
        (pa_ref[...] + pb_ref[...]) * inv_ref[...] + bl_ref[0:1, :]
        + r_ref[...], 0.0
    )
    y_ref[...] = jnp.dot(h, wl_ref[...], preferred_element_type=jnp.float32)
    rn_ref[...] = jnp.dot(h, wr_ref[...], preferred_element_type=jnp.float32)


def _stage_d_body(pa_ref, pb_ref, inv_ref, bl_ref, r_ref, w4_ref, b4_ref,
                  o_ref):
    h = jnp.maximum(
        (pa_ref[...] + pb_ref[...]) * inv_ref[...] + bl_ref[0:1, :]
        + r_ref[...], 0.0
    )
    o_ref[...] = (
        jnp.dot(h, w4_ref[...], preferred_element_type=jnp.float32)
        + b4_ref[0:1, :]
    )


def _rows(bs):
    return pl.BlockSpec((BR, bs), lambda i: (i, 0))


def _full(a, b):
    return pl.BlockSpec((a, b), lambda i: (0, 0))


def kernel(x, edge_index, Wl1, bl1, Wr1, Wl2, bl2, Wr2, Wl3, bl3, Wr3, W4, b4):
    xp = jnp.concatenate(
        [x, jnp.zeros((NPAD - N_NODES, C_IN), jnp.float32)], axis=0
    )
    epad = EPAD - E_EDGES
    srcp = jnp.concatenate(
        [edge_index[0], jnp.zeros((epad,), jnp.int32)]
    ).reshape(EROWS, CHUNK)
    dstp = jnp.concatenate(
        [edge_index[1], jnp.full((epad,), NPAD - 1, jnp.int32)]
    ).reshape(EROWS, CHUNK)

    bl1b = jnp.broadcast_to(bl1[None, :], (8, C_HID))
    bl2b = jnp.broadcast_to(bl2[None, :], (8, C_HID))
    bl3b = jnp.broadcast_to(bl3[None, :], (8, C_HID))
    b4b = jnp.broadcast_to(b4[None, :], (8, C_OUT))

    da, db = _sc_deg(dstp)

    y1, r1 = pl.pallas_call(
        _stage_a_body,
        grid=(NPAD // BR,),
        in_specs=[_rows(C_IN), _full(C_IN, C_HID), _full(C_IN, C_HID)],
        out_specs=[_rows(C_HID), _rows(C_HID)],
        out_shape=[jax.ShapeDtypeStruct((NPAD, C_HID), jnp.float32)] * 2,
    )(xp, Wl1, Wr1)

    a1, b1 = _sc_agg(y1, srcp, dstp, da, db)
    y2, r2, invd = pl.pallas_call(
        _stage_b_body,
        grid=(NPAD // BR,),
        in_specs=[_rows(C_HID), _rows(C_HID), _rows(DW), _rows(DW),
                  _full(8, C_HID), _rows(C_HID), _full(C_HID, C_HID),
                  _full(C_HID, C_HID)],
        out_specs=[_rows(C_HID), _rows(C_HID), _rows(C_HID)],
        out_shape=[jax.ShapeDtypeStruct((NPAD, C_HID), jnp.float32)] * 3,
    )(a1, b1, da, db, bl1b, r1, Wl2, Wr2)

    a2, b2 = _sc_agg(y2, srcp, dstp, da, db)
    y3, r3 = pl.pallas_call(
        _stage_c_body,
        grid=(NPAD // BR,),
        in_specs=[_rows(C_HID), _rows(C_HID), _rows(C_HID), _full(8, C_HID),
                  _rows(C_HID), _full(C_HID, C_HID), _full(C_HID, C_HID)],
        out_specs=[_rows(C_HID), _rows(C_HID)],
        out_shape=[jax.ShapeDtypeStruct((NPAD, C_HID), jnp.float32)] * 2,
    )(a2, b2, invd, bl2b, r2, Wl3, Wr3)

    a3, b3 = _sc_agg(y3, srcp, dstp, da, db)
    out = pl.pallas_call(
        _stage_d_body,
        grid=(NPAD // BR,),
        in_specs=[_rows(C_HID), _rows(C_HID), _rows(C_HID), _full(8, C_HID),
                  _rows(C_HID), _full(C_HID, C_OUT), _full(8, C_OUT)],
        out_specs=_rows(C_OUT),
        out_shape=jax.ShapeDtypeStruct((NPAD, C_OUT), jnp.float32),
    )(a3, b3, invd, bl3b, r3, W4, b4b)

    return out[:N_NODES - 1]


# speedup vs baseline: 1.2716x; 1.1756x over previous
import functools

import jax
import jax.numpy as jnp
from jax import lax
from jax.experimental import pallas as pl
from jax.experimental.pallas import tpu as pltpu
from jax.experimental.pallas import tpu_sc as plsc

N_NODES = 10000
E_EDGES = 320000
C_IN, C_HID, C_OUT = 128, 32, 8

NC, NS = 2, 16
NW = NC * NS
NPAD = 10240
RPS = NPAD // NS
CHUNK = 128
CPW = 80
EPAD = NW * CPW * CHUNK
EROWS = EPAD // CHUNK
ZROWS = 128
DW = 32
NB = 8

_MESH = plsc.VectorSubcoreMesh(
    core_axis_name="c", subcore_axis_name="s", num_cores=NC, num_subcores=NS
)
_SC_PARAMS = pltpu.CompilerParams(use_tc_tiling_on_sc=False)


def _fill2d(buf, rows, cols, value):
    segs = cols // 16

    def body(k, _):
        buf[k // segs, pl.ds((k % segs) * 16, 16)] = jnp.full(
            (16,), value, jnp.float32
        )
        return 0

    lax.fori_loop(0, rows * segs, body, 0)


@functools.partial(
    pl.kernel,
    out_type=[jax.ShapeDtypeStruct((NPAD, C_HID), jnp.float32)] * NC,
    mesh=_MESH,
    scratch_types=[
        pltpu.VMEM_SHARED((NPAD, C_HID), jnp.float32),
        pltpu.VMEM_SHARED((NPAD, C_HID), jnp.float32),
        pltpu.VMEM((ZROWS, C_HID), jnp.float32),
        pltpu.VMEM((CPW, CHUNK), jnp.int32),
        pltpu.VMEM((CPW, CHUNK), jnp.int32),
        pltpu.VMEM((NB, CHUNK, C_HID), jnp.float32),
        pltpu.SemaphoreType.DMA,
        pltpu.SemaphoreType.DMA,
        pltpu.SemaphoreType.DMA,
        pltpu.SemaphoreType.DMA,
        pltpu.SemaphoreType.DMA,
        pltpu.SemaphoreType.DMA,
        pltpu.SemaphoreType.DMA,
        pltpu.SemaphoreType.DMA,
        pltpu.SemaphoreType.DMA,
        pltpu.SemaphoreType.DMA,
        pltpu.SemaphoreType.DMA,
        pltpu.SemaphoreType.DMA,
        pltpu.SemaphoreType.DMA,
        pltpu.SemaphoreType.DMA,
        pltpu.SemaphoreType.DMA,
        pltpu.SemaphoreType.DMA,
    ],
    compiler_params=_SC_PARAMS,
)
def _sc_agg(table, srch, dsth, dga, dgb, outa, outb, acc, tbl, zrows, sidx,
            didx, rows, *sems):
    c = lax.axis_index("c")
    s = lax.axis_index("s")
    wid = s * NC + c
    gsem = sems[:NB]
    tsem = sems[NB:]
    LEAD = 4

    _fill2d(zrows, ZROWS, C_HID, 0.0)
    for k in range(RPS // ZROWS):
        pltpu.sync_copy(zrows, acc.at[pl.ds(s * RPS + k * ZROWS, ZROWS)])
    pltpu.sync_copy(srch.at[pl.ds(wid * CPW, CPW)], sidx)
    pltpu.sync_copy(dsth.at[pl.ds(wid * CPW, CPW)], didx)
    pltpu.sync_copy(table.at[pl.ds(s * RPS, RPS)], tbl.at[pl.ds(s * RPS, RPS)])
    plsc.subcore_barrier()

    def _wait_gather(b):
        pltpu.make_async_copy(tbl.at[sidx.at[0]], rows.at[b],
                              gsem[b]).wait()

    def _wait_scatter(b):
        pltpu.make_async_copy(rows.at[b], acc.at[didx.at[0]],
                              tsem[b]).wait()

    def _chunk(j, jj):
        b = j % NB
        _wait_gather(b)
        pltpu.async_copy(rows.at[b], acc.at[didx.at[jj]], tsem[b], add=True)

    for b in range(LEAD):
        pltpu.async_copy(tbl.at[sidx.at[b]], rows.at[b], gsem[b])

    for j in range(NB):
        _chunk(j, j)
        nxt = j + LEAD
        if nxt < NB:
            pass
        else:
            _wait_scatter(nxt % NB)
        pltpu.async_copy(tbl.at[sidx.at[nxt]], rows.at[nxt % NB],
                         gsem[nxt % NB])

    def body(g, _):
        for b in range(NB):
            j = NB * g + b
            _chunk(b, j)
            b2 = (b + LEAD) % NB
            _wait_scatter(b2)
            pltpu.async_copy(tbl.at[sidx.at[j + LEAD]], rows.at[b2],
                             gsem[b2])
        return 0

    lax.fori_loop(1, CPW // NB - 1, body, 0)

    for j in range(CPW - NB, CPW):
        _chunk(j % NB, j)
        nxt = j + LEAD
        if nxt < CPW:
            _wait_scatter(nxt % NB)
            pltpu.async_copy(tbl.at[sidx.at[nxt]], rows.at[nxt % NB],
                             gsem[nxt % NB])

    for b in range(NB):
        _wait_scatter(b)

    plsc.subcore_barrier()

    @pl.when(c == 0)
    def _():
        pltpu.sync_copy(acc.at[pl.ds(s * RPS, RPS)], outa.at[pl.ds(s * RPS, RPS)])

    @pl.when(c == 1)
    def _():
        pltpu.sync_copy(acc.at[pl.ds(s * RPS, RPS)], outb.at[pl.ds(s * RPS, RPS)])


@functools.partial(
    pl.kernel,
    out_type=[jax.ShapeDtypeStruct((NPAD, DW), jnp.float32)] * NC,
    mesh=_MESH,
    scratch_types=[
        pltpu.VMEM_SHARED((NPAD, DW), jnp.float32),
        pltpu.VMEM((ZROWS, DW), jnp.float32),
        pltpu.VMEM((CHUNK, DW), jnp.float32),
        pltpu.VMEM((CPW, CHUNK), jnp.int32),
    ],
    compiler_params=_SC_PARAMS,
)
def _sc_deg(dsth, outa, outb, dacc, zrows, ones, didx):
    c = lax.axis_index("c")
    s = lax.axis_index("s")
    wid = s * NC + c

    _fill2d(zrows, ZROWS, DW, 0.0)
    _fill2d(ones, CHUNK, DW, 1.0)
    for k in range(RPS // ZROWS):
        pltpu.sync_copy(zrows, dacc.at[pl.ds(s * RPS + k * ZROWS, ZROWS)])
    pltpu.sync_copy(dsth.at[pl.ds(wid * CPW, CPW)], didx)
    plsc.subcore_barrier()

    def body(j, _):
        pltpu.sync_copy(ones, dacc.at[didx.at[j]], add=True)
        return 0

    lax.fori_loop(0, CPW, body, 0)

    plsc.subcore_barrier()

    @pl.when(c == 0)
    def _():
        pltpu.sync_copy(dacc.at[pl.ds(s * RPS, RPS)], outa.at[pl.ds(s * RPS, RPS)])

    @pl.when(c == 1)
    def _():
        pltpu.sync_copy(dacc.at[pl.ds(s * RPS, RPS)], outb.at[pl.ds(s * RPS, RPS)])


NPACK = NPAD // 4
PL = 4 * C_HID
BR = 512


def _stage_a_body(x_ref, wl_ref, wr_ref, y_ref, r_ref):
    xb = x_ref[...]
    y_ref[...] = jnp.dot(xb, wl_ref[...], preferred_element_type=jnp.float32)
    r_ref[...] = jnp.dot(xb, wr_ref[...], preferred_element_type=jnp.float32)


def _stage_b_body(pa_ref, pb_ref, da_ref, db_ref, bl_ref, r_ref, wl_ref,
                  wr_ref, y_ref, rn_ref, inv_ref):
    inv = 1.0 / jnp.maximum(da_ref[...] + db_ref[...], 1.0)
    h = jnp.maximum(
        (pa_ref[...] + pb_ref[...]) * inv + bl_ref[0:1, :] + r_ref[...], 0.0
    )
    y_ref[...] = jnp.dot(h, wl_ref[...], preferred_element_type=jnp.float32)
    rn_ref[...] = jnp.dot(h, wr_ref[...], preferred_element_type=jnp.float32)
    inv_ref[...] = inv


def _stage_c_body(pa_ref, pb_ref, inv_ref, bl_ref, r_ref, wl_ref, wr_ref,
                  y_ref, rn_ref):
    h = jnp.maximum(
        (pa_ref[...] + pb_ref[...]) * inv_ref[...] + bl_ref[0:1, :]
        + r_ref[...], 0.0
    )
    y_ref[...] = jnp.dot(h, wl_ref[...], preferred_element_type=jnp.float32)
    rn_ref[...] = jnp.dot(h, wr_ref[...], preferred_element_type=jnp.float32)


def _stage_d_body(pa_ref, pb_ref, inv_ref, bl_ref, r_ref, w4_ref, b4_ref,
                  o_ref):
    h = jnp.maximum(
        (pa_ref[...] + pb_ref[...]) * inv_ref[...] + bl_ref[0:1, :]
        + r_ref[...], 0.0
    )
    o_ref[...] = (
        jnp.dot(h, w4_ref[...], preferred_element_type=jnp.float32)
        + b4_ref[0:1, :]
    )


def _bd4(w):
    return jax.scipy.linalg.block_diag(w, w, w, w)


def _rows(bs):
    return pl.BlockSpec((BR, bs), lambda i: (i, 0))


def _full(a, b):
    return pl.BlockSpec((a, b), lambda i: (0, 0))


def kernel(x, edge_index, Wl1, bl1, Wr1, Wl2, bl2, Wr2, Wl3, bl3, Wr3, W4, b4):
    xp = jnp.concatenate(
        [x, jnp.zeros((NPAD - N_NODES, C_IN), jnp.float32)], axis=0
    ).reshape(NPACK, 4 * C_IN)
    epad = EPAD - E_EDGES
    srcp = jnp.concatenate(
        [edge_index[0], jnp.zeros((epad,), jnp.int32)]
    ).reshape(EROWS, CHUNK)
    dstp = jnp.concatenate(
        [edge_index[1], jnp.full((epad,), NPAD - 1, jnp.int32)]
    ).reshape(EROWS, CHUNK)

    bl1b = jnp.broadcast_to(jnp.tile(bl1, 4)[None, :], (8, PL))
    bl2b = jnp.broadcast_to(jnp.tile(bl2, 4)[None, :], (8, PL))
    bl3b = jnp.broadcast_to(jnp.tile(bl3, 4)[None, :], (8, PL))
    b4b = jnp.broadcast_to(jnp.tile(b4, 4)[None, :], (8, 4 * C_OUT))

    da, db = _sc_deg(dstp)
    dap = da.reshape(NPACK, PL)
    dbp = db.reshape(NPACK, PL)

    y1, r1 = pl.pallas_call(
        _stage_a_body,
        grid=(NPACK // BR,),
        in_specs=[_rows(4 * C_IN), _full(4 * C_IN, PL), _full(4 * C_IN, PL)],
        out_specs=[_rows(PL), _rows(PL)],
        out_shape=[jax.ShapeDtypeStruct((NPACK, PL), jnp.float32)] * 2,
    )(xp, _bd4(Wl1), _bd4(Wr1))

    a1, b1 = _sc_agg(y1.reshape(NPAD, C_HID), srcp, dstp, da, db)
    y2, r2, invd = pl.pallas_call(
        _stage_b_body,
        grid=(NPACK // BR,),
        in_specs=[_rows(PL), _rows(PL), _rows(PL), _rows(PL),
                  _full(8, PL), _rows(PL), _full(PL, PL), _full(PL, PL)],
        out_specs=[_rows(PL), _rows(PL), _rows(PL)],
        out_shape=[jax.ShapeDtypeStruct((NPACK, PL), jnp.float32)] * 3,
    )(a1.reshape(NPACK, PL), b1.reshape(NPACK, PL), dap, dbp, bl1b, r1,
      _bd4(Wl2), _bd4(Wr2))

    a2, b2 = _sc_agg(y2.reshape(NPAD, C_HID), srcp, dstp, da, db)
    y3, r3 = pl.pallas_call(
        _stage_c_body,
        grid=(NPACK // BR,),
        in_specs=[_rows(PL), _rows(PL), _rows(PL), _full(8, PL),
                  _rows(PL), _full(PL, PL), _full(PL, PL)],
        out_specs=[_rows(PL), _rows(PL)],
        out_shape=[jax.ShapeDtypeStruct((NPACK, PL), jnp.float32)] * 2,
    )(a2.reshape(NPACK, PL), b2.reshape(NPACK, PL), invd, bl2b, r2,
      _bd4(Wl3), _bd4(Wr3))

    a3, b3 = _sc_agg(y3.reshape(NPAD, C_HID), srcp, dstp, da, db)
    out = pl.pallas_call(
        _stage_d_body,
        grid=(NPACK // BR,),
        in_specs=[_rows(PL), _rows(PL), _rows(PL), _full(8, PL),
                  _rows(PL), _full(PL, 4 * C_OUT), _full(8, 4 * C_OUT)],
        out_specs=_rows(4 * C_OUT),
        out_shape=jax.ShapeDtypeStruct((NPACK, 4 * C_OUT), jnp.float32),
    )(a3.reshape(NPACK, PL), b3.reshape(NPACK, PL), invd, bl3b, r3,
      _bd4(W4), b4b)

    return out.reshape(NPAD, C_OUT)[:N_NODES - 1]
